# baseline (speedup 1.0000x reference)
import functools

import jax
import jax.numpy as jnp
from jax import lax
from jax.experimental import pallas as pl
from jax.experimental.pallas import tpu as pltpu
from jax.experimental.pallas import tpu_sc as plsc

NC = 2
NS = 16
NW = NC * NS
TRASH = 1 << 20


def _redirect_body(nout, cur_a_ref, nxt_a_ref, out_a_ref):
    cur = cur_a_ref[...]
    nxt = nxt_a_ref[...]
    succ = jnp.concatenate([cur[1:], nxt[:1]])
    trash = nout + (cur & (TRASH - 1))
    out_a_ref[...] = jnp.where(succ == cur, trash, cur)


def _make_scatter(n, chunk, nbuf):
    nchunks = n // chunk
    per_w = nchunks // NW
    extra_w = nchunks - per_w * NW
    mesh = plsc.VectorSubcoreMesh(
        core_axis_name="c", subcore_axis_name="s",
        num_cores=NC, num_subcores=NS)

    @functools.partial(
        pl.kernel,
        out_type=(),
        mesh=mesh,
        scratch_types=(
            [pltpu.VMEM((chunk,), jnp.int32) for _ in range(nbuf)]
            + [pltpu.VMEM((chunk,), jnp.float32) for _ in range(nbuf)]
            + [pltpu.SemaphoreType.DMA, pltpu.SemaphoreType.DMA]
        ),
    )
    def scatter_kernel(addr_hbm, val_hbm, out_ref, *scratch):
        addr_v = scratch[:nbuf]
        val_v = scratch[nbuf: 2 * nbuf]
        lsem, ssem = scratch[2 * nbuf], scratch[2 * nbuf + 1]
        wid = lax.axis_index("s") * NC + lax.axis_index("c")
        has_extra = wid < extra_w

        def start_load(k):
            base = (wid + k * NW) * chunk
            buf = k % nbuf
            return (
                pltpu.async_copy(
                    addr_hbm.at[pl.ds(base, chunk)], addr_v[buf], lsem),
                pltpu.async_copy(
                    val_hbm.at[pl.ds(base, chunk)], val_v[buf], lsem),
            )

        def start_scatter(k):
            buf = k % nbuf
            return pltpu.async_copy(
                val_v[buf], out_ref.at[addr_v[buf]], ssem)

        loads = {0: start_load(0)}
        scats = {}
        for k in range(per_w):
            if k + 1 - nbuf >= 0:
                scats.pop(k + 1 - nbuf).wait()
            if k + 1 < per_w:
                loads[k + 1] = start_load(k + 1)
            for h in loads.pop(k):
                h.wait()
            scats[k] = start_scatter(k)
        for k in sorted(scats):
            scats[k].wait()

        @pl.when(has_extra)
        def _tail():
            ha, hv = start_load(per_w)
            ha.wait()
            hv.wait()
            start_scatter(per_w).wait()

    return scatter_kernel


def kernel(self_tensor, index, updates, axis):
    m, d = self_tensor.shape
    b = index.shape[0]
    n = b * d
    nout = m * d

    addr = (index.astype(jnp.int32) * d
            + jnp.arange(d, dtype=jnp.int32)[None, :]).reshape(-1)
    vals = updates.reshape(-1)
    sa, sv = lax.sort((addr, vals), dimension=0, num_keys=1,
                      is_stable=False)

    blr = 256000
    assert n % blr == 0
    nblk = n // blr
    sa_p = jnp.pad(sa, (0, blr), constant_values=-1)
    sa2 = pl.pallas_call(
        functools.partial(_redirect_body, nout),
        grid=(nblk,),
        in_specs=[
            pl.BlockSpec((blr,), lambda c: (c,)),
            pl.BlockSpec((blr,), lambda c: (c + 1,)),
        ],
        out_specs=pl.BlockSpec((blr,), lambda c: (c,)),
        out_shape=jax.ShapeDtypeStruct((n,), jnp.int32),
    )(sa_p, sa_p)

    out_flat = jnp.concatenate(
        [self_tensor.reshape(nout),
         jnp.zeros((TRASH,), jnp.float32)])
    out_ref = jax.new_ref(out_flat)
    _make_scatter(n, 2048, 3)(sa2, sv, out_ref)
    return out_ref[...][:nout].reshape(m, d)
